# Initial kernel scaffold; baseline (speedup 1.0000x reference)
import jax
import jax.numpy as jnp
from jax.experimental import pallas as pl


def kernel(x, W1, b1, W2):
    raise NotImplementedError("write your pallas kernel here")


# speedup vs baseline: 1.0067x; 1.0067x over previous
import functools

import jax
import jax.numpy as jnp
from jax.experimental import pallas as pl
from jax.experimental.pallas import tpu as pltpu

N_TOKENS = 32768
INPUT_SIZE = 4096
HIDDEN_SIZE = 1024
NUM_EXPERTS = 64
TOP_K = 8

BT = 256


def _gating_body(x_ref, w1_ref, b1_ref, w2_ref, idx_ref, gates_ref, probs_ref):
    h = jnp.dot(x_ref[...], w1_ref[...], preferred_element_type=jnp.float32)
    h = h + b1_ref[...]
    h = 0.5 * h * (1.0 + jax.lax.erf(h * 0.7071067811865476))
    logits = jnp.dot(h, w2_ref[...], preferred_element_type=jnp.float32)
    m = jnp.max(logits, axis=-1, keepdims=True)
    e = jnp.exp(logits - m)
    probs = e / jnp.sum(e, axis=-1, keepdims=True)
    probs_ref[...] = probs

    iota = jax.lax.broadcasted_iota(jnp.int32, probs.shape, 1)
    p = probs
    gate_cols = []
    idx_cols = []
    for _ in range(TOP_K):
        mk = jnp.max(p, axis=-1, keepdims=True)
        is_max = p == mk
        ik = jnp.min(jnp.where(is_max, iota, NUM_EXPERTS), axis=-1, keepdims=True)
        gate_cols.append(mk)
        idx_cols.append(ik)
        p = jnp.where(iota == ik, -jnp.inf, p)
    gates_ref[...] = jnp.concatenate(gate_cols, axis=-1)
    idx_ref[...] = jnp.concatenate(idx_cols, axis=-1)


@jax.jit
def kernel(x, W1, b1, W2):
    b1r = b1.reshape(1, HIDDEN_SIZE)
    grid = (N_TOKENS // BT,)
    out = pl.pallas_call(
        _gating_body,
        grid=grid,
        in_specs=[
            pl.BlockSpec((BT, INPUT_SIZE), lambda i: (i, 0)),
            pl.BlockSpec((INPUT_SIZE, HIDDEN_SIZE), lambda i: (0, 0)),
            pl.BlockSpec((1, HIDDEN_SIZE), lambda i: (0, 0)),
            pl.BlockSpec((HIDDEN_SIZE, NUM_EXPERTS), lambda i: (0, 0)),
        ],
        out_specs=[
            pl.BlockSpec((BT, TOP_K), lambda i: (i, 0)),
            pl.BlockSpec((BT, TOP_K), lambda i: (i, 0)),
            pl.BlockSpec((BT, NUM_EXPERTS), lambda i: (i, 0)),
        ],
        out_shape=[
            jax.ShapeDtypeStruct((N_TOKENS, TOP_K), jnp.int32),
            jax.ShapeDtypeStruct((N_TOKENS, TOP_K), jnp.float32),
            jax.ShapeDtypeStruct((N_TOKENS, NUM_EXPERTS), jnp.float32),
        ],
        compiler_params=pltpu.CompilerParams(
            dimension_semantics=("parallel",),
        ),
    )(x, W1, b1r, W2)
    return (out[0], out[1], out[2])


# speedup vs baseline: 1.6159x; 1.6050x over previous
import jax
import jax.numpy as jnp
from jax.experimental import pallas as pl
from jax.experimental.pallas import tpu as pltpu

N_TOKENS = 32768
INPUT_SIZE = 4096
HIDDEN_SIZE = 1024
NUM_EXPERTS = 64
TOP_K = 8

BT = 256
NB = N_TOKENS // BT


def _gating_body(x_ref, w1_ref, b1_ref, w2_ref, idx_ref, gates_ref, probs_ref,
                 logits_ref):
    logits = logits_ref[...]
    m = jnp.max(logits, axis=-1, keepdims=True)
    e = jnp.exp(logits - m)
    probs = e / jnp.sum(e, axis=-1, keepdims=True)
    probs_ref[...] = probs

    iota = jax.lax.broadcasted_iota(jnp.int32, probs.shape, 1)
    p = probs
    gate_cols = []
    idx_cols = []
    for _ in range(TOP_K):
        mk = jnp.max(p, axis=-1, keepdims=True)
        is_max = p == mk
        ik = jnp.min(jnp.where(is_max, iota, NUM_EXPERTS), axis=-1, keepdims=True)
        gate_cols.append(mk)
        idx_cols.append(ik)
        p = jnp.where(iota == ik, -jnp.inf, p)
    gates_ref[...] = jnp.concatenate(gate_cols, axis=-1)
    idx_ref[...] = jnp.concatenate(idx_cols, axis=-1)

    h = jnp.dot(x_ref[...], w1_ref[...], preferred_element_type=jnp.float32)
    h = h + b1_ref[...]
    h = 0.5 * h * (1.0 + jax.lax.erf(h * 0.7071067811865476))
    logits_ref[...] = jnp.dot(h, w2_ref[...], preferred_element_type=jnp.float32)


@jax.jit
def kernel(x, W1, b1, W2):
    b1r = b1.reshape(1, HIDDEN_SIZE)
    grid = (NB + 1,)
    out = pl.pallas_call(
        _gating_body,
        grid=grid,
        in_specs=[
            pl.BlockSpec((BT, INPUT_SIZE), lambda i: (jnp.minimum(i, NB - 1), 0)),
            pl.BlockSpec((INPUT_SIZE, HIDDEN_SIZE), lambda i: (0, 0)),
            pl.BlockSpec((1, HIDDEN_SIZE), lambda i: (0, 0)),
            pl.BlockSpec((HIDDEN_SIZE, NUM_EXPERTS), lambda i: (0, 0)),
        ],
        out_specs=[
            pl.BlockSpec((BT, TOP_K), lambda i: (jnp.maximum(i - 1, 0), 0)),
            pl.BlockSpec((BT, TOP_K), lambda i: (jnp.maximum(i - 1, 0), 0)),
            pl.BlockSpec((BT, NUM_EXPERTS), lambda i: (jnp.maximum(i - 1, 0), 0)),
        ],
        out_shape=[
            jax.ShapeDtypeStruct((N_TOKENS, TOP_K), jnp.int32),
            jax.ShapeDtypeStruct((N_TOKENS, TOP_K), jnp.float32),
            jax.ShapeDtypeStruct((N_TOKENS, NUM_EXPERTS), jnp.float32),
        ],
        scratch_shapes=[pltpu.VMEM((BT, NUM_EXPERTS), jnp.float32)],
        compiler_params=pltpu.CompilerParams(
            dimension_semantics=("arbitrary",),
        ),
    )(x, W1, b1r, W2)
    return (out[0], out[1], out[2])


# speedup vs baseline: 1.7300x; 1.0706x over previous
import jax
import jax.numpy as jnp
from jax.experimental import pallas as pl
from jax.experimental.pallas import tpu as pltpu

N_TOKENS = 32768
INPUT_SIZE = 4096
HIDDEN_SIZE = 1024
NUM_EXPERTS = 64
TOP_K = 8

BT = 1024
NB = N_TOKENS // BT


def _gating_body(x_ref, w1_ref, b1_ref, w2_ref, idx_ref, gates_ref, probs_ref,
                 logits_ref):
    CHUNK = 128
    for c in range(BT // CHUNK):
        sl = pl.ds(c * CHUNK, CHUNK)
        logits = logits_ref[sl, :]
        m = jnp.max(logits, axis=-1, keepdims=True)
        e = jnp.exp(logits - m)
        r = 1.0 / jnp.sum(e, axis=-1, keepdims=True)
        probs_ref[sl, :] = e * r

        iota = jax.lax.broadcasted_iota(jnp.int32, logits.shape, 1).astype(jnp.float32)
        p = logits
        gate_cols = []
        idx_cols = []
        for _ in range(TOP_K):
            mk = jnp.max(p, axis=-1, keepdims=True)
            is_max = p == mk
            ik = jnp.min(jnp.where(is_max, iota, float(NUM_EXPERTS)),
                         axis=-1, keepdims=True)
            gate_cols.append(jnp.exp(mk - m) * r)
            idx_cols.append(ik)
            p = jnp.where(iota == ik, -jnp.inf, p)
        gates_ref[sl, :] = jnp.concatenate(gate_cols, axis=-1)
        idx_ref[sl, :] = jnp.concatenate(idx_cols, axis=-1).astype(jnp.int32)

    h = jnp.dot(x_ref[...], w1_ref[...], preferred_element_type=jnp.float32)
    h = h + b1_ref[...]
    h = 0.5 * h * (1.0 + jax.lax.erf(h * 0.7071067811865476))
    logits_ref[...] = jnp.dot(h, w2_ref[...], preferred_element_type=jnp.float32)


@jax.jit
def kernel(x, W1, b1, W2):
    b1r = b1.reshape(1, HIDDEN_SIZE)
    grid = (NB + 1,)
    out = pl.pallas_call(
        _gating_body,
        grid=grid,
        in_specs=[
            pl.BlockSpec((BT, INPUT_SIZE), lambda i: (jnp.minimum(i, NB - 1), 0)),
            pl.BlockSpec((INPUT_SIZE, HIDDEN_SIZE), lambda i: (0, 0)),
            pl.BlockSpec((1, HIDDEN_SIZE), lambda i: (0, 0)),
            pl.BlockSpec((HIDDEN_SIZE, NUM_EXPERTS), lambda i: (0, 0)),
        ],
        out_specs=[
            pl.BlockSpec((BT, TOP_K), lambda i: (jnp.maximum(i - 1, 0), 0)),
            pl.BlockSpec((BT, TOP_K), lambda i: (jnp.maximum(i - 1, 0), 0)),
            pl.BlockSpec((BT, NUM_EXPERTS), lambda i: (jnp.maximum(i - 1, 0), 0)),
        ],
        out_shape=[
            jax.ShapeDtypeStruct((N_TOKENS, TOP_K), jnp.int32),
            jax.ShapeDtypeStruct((N_TOKENS, TOP_K), jnp.float32),
            jax.ShapeDtypeStruct((N_TOKENS, NUM_EXPERTS), jnp.float32),
        ],
        scratch_shapes=[pltpu.VMEM((BT, NUM_EXPERTS), jnp.float32)],
        compiler_params=pltpu.CompilerParams(
            dimension_semantics=("arbitrary",),
        ),
    )(x, W1, b1r, W2)
    return (out[0], out[1], out[2])


# speedup vs baseline: 1.7378x; 1.0045x over previous
import jax
import jax.numpy as jnp
from jax.experimental import pallas as pl
from jax.experimental.pallas import tpu as pltpu

N_TOKENS = 32768
INPUT_SIZE = 4096
HIDDEN_SIZE = 1024
NUM_EXPERTS = 64
TOP_K = 8

BT = 1024
NB = N_TOKENS // BT


def _gating_body(x_ref, w1_ref, b1_ref, w2_ref, idx_ref, gates_ref, probs_ref,
                 logits_ref):
    CHUNK = 64
    for c in range(BT // CHUNK):
        sl = pl.ds(c * CHUNK, CHUNK)
        logits = logits_ref[sl, :]
        m = jnp.max(logits, axis=-1, keepdims=True)
        e = jnp.exp(logits - m)
        r = 1.0 / jnp.sum(e, axis=-1, keepdims=True)
        probs_ref[sl, :] = e * r

        iota = jax.lax.broadcasted_iota(jnp.int32, logits.shape, 1).astype(jnp.float32)
        p = logits
        gate_cols = []
        idx_cols = []
        for _ in range(TOP_K):
            mk = jnp.max(p, axis=-1, keepdims=True)
            is_max = p == mk
            ik = jnp.min(jnp.where(is_max, iota, float(NUM_EXPERTS)),
                         axis=-1, keepdims=True)
            gate_cols.append(jnp.exp(mk - m) * r)
            idx_cols.append(ik)
            p = jnp.where(iota == ik, -jnp.inf, p)
        gates_ref[sl, :] = jnp.concatenate(gate_cols, axis=-1)
        idx_ref[sl, :] = jnp.concatenate(idx_cols, axis=-1).astype(jnp.int32)

    h = jnp.dot(x_ref[...], w1_ref[...], preferred_element_type=jnp.float32)
    h = h + b1_ref[...]
    h = 0.5 * h * (1.0 + jax.lax.erf(h * 0.7071067811865476))
    logits_ref[...] = jnp.dot(h, w2_ref[...], preferred_element_type=jnp.float32)


@jax.jit
def kernel(x, W1, b1, W2):
    b1r = b1.reshape(1, HIDDEN_SIZE)
    grid = (NB + 1,)
    out = pl.pallas_call(
        _gating_body,
        grid=grid,
        in_specs=[
            pl.BlockSpec((BT, INPUT_SIZE), lambda i: (jnp.minimum(i, NB - 1), 0)),
            pl.BlockSpec((INPUT_SIZE, HIDDEN_SIZE), lambda i: (0, 0)),
            pl.BlockSpec((1, HIDDEN_SIZE), lambda i: (0, 0)),
            pl.BlockSpec((HIDDEN_SIZE, NUM_EXPERTS), lambda i: (0, 0)),
        ],
        out_specs=[
            pl.BlockSpec((BT, TOP_K), lambda i: (jnp.maximum(i - 1, 0), 0)),
            pl.BlockSpec((BT, TOP_K), lambda i: (jnp.maximum(i - 1, 0), 0)),
            pl.BlockSpec((BT, NUM_EXPERTS), lambda i: (jnp.maximum(i - 1, 0), 0)),
        ],
        out_shape=[
            jax.ShapeDtypeStruct((N_TOKENS, TOP_K), jnp.int32),
            jax.ShapeDtypeStruct((N_TOKENS, TOP_K), jnp.float32),
            jax.ShapeDtypeStruct((N_TOKENS, NUM_EXPERTS), jnp.float32),
        ],
        scratch_shapes=[pltpu.VMEM((BT, NUM_EXPERTS), jnp.float32)],
        compiler_params=pltpu.CompilerParams(
            dimension_semantics=("arbitrary",),
        ),
    )(x, W1, b1r, W2)
    return (out[0], out[1], out[2])


# speedup vs baseline: 1.7703x; 1.0187x over previous
import jax
import jax.numpy as jnp
from jax.experimental import pallas as pl
from jax.experimental.pallas import tpu as pltpu

N_TOKENS = 32768
INPUT_SIZE = 4096
HIDDEN_SIZE = 1024
NUM_EXPERTS = 64
TOP_K = 8

BT = 1024
NB = N_TOKENS // BT


def _gating_body(x_ref, w1_ref, b1_ref, w2_ref, idx_ref, gates_ref, probs_ref,
                 logits_ref):
    CHUNK = 128
    for c in range(BT // CHUNK):
        sl = pl.ds(c * CHUNK, CHUNK)
        logits = logits_ref[sl, :]
        m = jnp.max(logits, axis=-1, keepdims=True)
        e = jnp.exp(logits - m)
        r = 1.0 / jnp.sum(e, axis=-1, keepdims=True)
        probs_ref[sl, :] = e * r

        iota = jax.lax.broadcasted_iota(jnp.int32, logits.shape, 1).astype(jnp.float32)
        p = e * r
        gate_cols = []
        idx_cols = []
        for _ in range(TOP_K):
            mk = jnp.max(p, axis=-1, keepdims=True)
            is_max = p == mk
            ik = jnp.min(jnp.where(is_max, iota, float(NUM_EXPERTS)),
                         axis=-1, keepdims=True)
            gate_cols.append(mk)
            idx_cols.append(ik)
            p = jnp.where(iota == ik, -1.0, p)
        gates_ref[sl, :] = jnp.concatenate(gate_cols, axis=-1)
        idx_ref[sl, :] = jnp.concatenate(idx_cols, axis=-1).astype(jnp.int32)

    h = jnp.dot(x_ref[...], w1_ref[...], preferred_element_type=jnp.float32)
    h = h + b1_ref[...]
    h = 0.5 * h * (1.0 + jax.lax.erf(h * 0.7071067811865476))
    logits_ref[...] = jnp.dot(h, w2_ref[...], preferred_element_type=jnp.float32)


@jax.jit
def kernel(x, W1, b1, W2):
    b1r = b1.reshape(1, HIDDEN_SIZE)
    grid = (NB + 1,)
    out = pl.pallas_call(
        _gating_body,
        grid=grid,
        in_specs=[
            pl.BlockSpec((BT, INPUT_SIZE), lambda i: (jnp.minimum(i, NB - 1), 0)),
            pl.BlockSpec((INPUT_SIZE, HIDDEN_SIZE), lambda i: (0, 0)),
            pl.BlockSpec((1, HIDDEN_SIZE), lambda i: (0, 0)),
            pl.BlockSpec((HIDDEN_SIZE, NUM_EXPERTS), lambda i: (0, 0)),
        ],
        out_specs=[
            pl.BlockSpec((BT, TOP_K), lambda i: (jnp.maximum(i - 1, 0), 0)),
            pl.BlockSpec((BT, TOP_K), lambda i: (jnp.maximum(i - 1, 0), 0)),
            pl.BlockSpec((BT, NUM_EXPERTS), lambda i: (jnp.maximum(i - 1, 0), 0)),
        ],
        out_shape=[
            jax.ShapeDtypeStruct((N_TOKENS, TOP_K), jnp.int32),
            jax.ShapeDtypeStruct((N_TOKENS, TOP_K), jnp.float32),
            jax.ShapeDtypeStruct((N_TOKENS, NUM_EXPERTS), jnp.float32),
        ],
        scratch_shapes=[pltpu.VMEM((BT, NUM_EXPERTS), jnp.float32)],
        compiler_params=pltpu.CompilerParams(
            dimension_semantics=("arbitrary",),
        ),
    )(x, W1, b1r, W2)
    return (out[0], out[1], out[2])
